# baseline (speedup 1.0000x reference)
import functools

import jax
import jax.numpy as jnp
from jax import lax
from jax.experimental import pallas as pl
from jax.experimental.pallas import tpu as pltpu
from jax.experimental.pallas import tpu_sc as plsc

_D = 32
_LANES = 16
_BC = 49152
_T_SPLIT = 12
_SC_END = 983040
_CW = 768


def _phase1_tc(ut_ref, it_ref, ub_ref, ib_ref, wu_ref, wi_ref, su_ref, si_ref):
    su_ref[...] = jnp.sum(ut_ref[...] * wu_ref[...], axis=0) + ub_ref[0, :]
    si_ref[...] = jnp.sum(it_ref[...] * wi_ref[...], axis=0) + ib_ref[0, :]


def _phase1_sc(cols_per_tile,
               ut_hbm, it_hbm, ubt_hbm, ibt_hbm, wb_hbm,
               su_hbm, si_hbm,
               ubuf0, ubuf1, ibuf0, ibuf1, ubb0, ubb1, ibb0, ibb1,
               wb_v, ou_v, oi_v,
               sem0, sem1):
    wid = lax.axis_index("s") * 2 + lax.axis_index("c")
    base = _T_SPLIT * _BC + wid * cols_per_tile

    pltpu.sync_copy(wb_hbm, wb_v)
    wvecs = [wb_v[pl.ds(16 * k, 16)] for k in range(4)]

    def wsc(d):
        return wvecs[d // 16][d % 16]

    nch = cols_per_tile // _CW
    ubufs = [ubuf0, ubuf1]
    ibufs = [ibuf0, ibuf1]
    ubbs = [ubb0, ubb1]
    ibbs = [ibb0, ibb1]
    sems = [sem0, sem1]

    def start(c):
        c0 = base + c * _CW
        p = c % 2
        return [
            pltpu.async_copy(ut_hbm.at[:, pl.ds(c0, _CW)], ubufs[p], sems[p]),
            pltpu.async_copy(it_hbm.at[:, pl.ds(c0, _CW)], ibufs[p], sems[p]),
            pltpu.async_copy(ubt_hbm.at[:, pl.ds(c0, _CW)], ubbs[p], sems[p]),
            pltpu.async_copy(ibt_hbm.at[:, pl.ds(c0, _CW)], ibbs[p], sems[p]),
        ]

    pending = {0: start(0)}
    for c in range(nch):
        if c + 1 < nch:
            pending[c + 1] = start(c + 1)
        for cp in pending.pop(c):
            cp.wait()
        p = c % 2
        ub, ib, ubb, ibb = ubufs[p], ibufs[p], ubbs[p], ibbs[p]

        def tree_sum(vs):
            while len(vs) > 1:
                nxt = [vs[k] + vs[k + 1] for k in range(0, len(vs) - 1, 2)]
                if len(vs) % 2:
                    nxt.append(vs[-1])
                vs = nxt
            return vs[0]

        def grp(g, carry):
            o = g * _LANES
            pu = [ubb[0, pl.ds(o, _LANES)]]
            pi = [ibb[0, pl.ds(o, _LANES)]]
            for d in range(_D):
                pu.append(ub[d, pl.ds(o, _LANES)] * wsc(d))
            for d in range(_D):
                pi.append(ib[d, pl.ds(o, _LANES)] * wsc(_D + d))
            oo = c * _CW + o
            ou_v[pl.ds(oo, _LANES)] = tree_sum(pu)
            oi_v[pl.ds(oo, _LANES)] = tree_sum(pi)
            return carry

        lax.fori_loop(0, _CW // _LANES, grp, 0)

    pltpu.sync_copy(ou_v, su_hbm.at[pl.ds(base, cols_per_tile)])
    pltpu.sync_copy(oi_v, si_hbm.at[pl.ds(base, cols_per_tile)])


def _phase2(b_per_w,
            uid_hbm, iid_hbm, su_hbm, si_hbm, su2_hbm, si2_hbm, b16_hbm,
            out_hbm,
            uid_v, iid_v, sug_v, sig_v, sug2_v, sig2_v, b_v, out_v, sem):
    wid = lax.axis_index("s") * 2 + lax.axis_index("c")
    base = wid * b_per_w

    pltpu.sync_copy(uid_hbm.at[pl.ds(base, b_per_w)], uid_v)
    pltpu.sync_copy(iid_hbm.at[pl.ds(base, b_per_w)], iid_v)
    pltpu.sync_copy(b16_hbm, b_v)

    cps = [
        pltpu.async_copy(su_hbm.at[uid_v], sug_v, sem),
        pltpu.async_copy(si_hbm.at[iid_v], sig_v, sem),
        pltpu.async_copy(su2_hbm.at[uid_v], sug2_v, sem),
        pltpu.async_copy(si2_hbm.at[iid_v], sig2_v, sem),
    ]
    for cp in cps:
        cp.wait()

    b_vec = b_v[...]
    lo = _T_SPLIT * _BC

    def blk(i, carry):
        rbase = i * _LANES
        sl = pl.ds(rbase, _LANES)
        uid = uid_v[sl]
        iid = iid_v[sl]
        u_tc = jnp.logical_or(uid < lo, uid >= _SC_END)
        i_tc = jnp.logical_or(iid < lo, iid >= _SC_END)
        su = jnp.where(u_tc, sug_v[sl], sug2_v[sl])
        si = jnp.where(i_tc, sig_v[sl], sig2_v[sl])
        out_v[sl] = jnp.clip(su + si + b_vec, 0.5, 5.0)
        return carry

    lax.fori_loop(0, b_per_w // _LANES, blk, 0)

    pltpu.sync_copy(out_v, out_hbm.at[pl.ds(base, b_per_w)])


def kernel(user_ids, item_ids, user_emb, item_emb, user_bias_tab, item_bias_tab, W, b):
    batch = user_ids.shape[0]
    n_workers = 32
    b_per_w = batch // n_workers
    n_rows = user_emb.shape[0]

    ut = user_emb.T
    it = item_emb.T
    ub_t = user_bias_tab.T
    ib_t = item_bias_tab.T
    wu = W[0, :_D].reshape(_D, 1)
    wi = W[0, _D:].reshape(_D, 1)
    wb = jnp.zeros((64,), jnp.float32).at[:2 * _D].set(W.reshape(-1))
    b16 = jnp.full((_LANES,), b[0], jnp.float32)

    nb = (n_rows + _BC - 1) // _BC
    tc_grid = _T_SPLIT + 1

    def colmap(j):
        return (0, jnp.where(j < _T_SPLIT, j, nb - 1))

    def outmap(j):
        return (jnp.where(j < _T_SPLIT, j, nb - 1),)

    su, si = pl.pallas_call(
        _phase1_tc,
        grid=(tc_grid,),
        in_specs=[
            pl.BlockSpec((_D, _BC), colmap),
            pl.BlockSpec((_D, _BC), colmap),
            pl.BlockSpec((1, _BC), colmap),
            pl.BlockSpec((1, _BC), colmap),
            pl.BlockSpec((_D, 1), lambda j: (0, 0)),
            pl.BlockSpec((_D, 1), lambda j: (0, 0)),
        ],
        out_specs=[
            pl.BlockSpec((_BC,), outmap),
            pl.BlockSpec((_BC,), outmap),
        ],
        out_shape=[
            jax.ShapeDtypeStruct((n_rows,), jnp.float32),
            jax.ShapeDtypeStruct((n_rows,), jnp.float32),
        ],
    )(ut, it, ub_t, ib_t, wu, wi)

    cols_per_tile = (_SC_END - _T_SPLIT * _BC) // n_workers
    mesh = plsc.VectorSubcoreMesh(core_axis_name="c", subcore_axis_name="s")
    su2, si2 = pl.kernel(
        functools.partial(_phase1_sc, cols_per_tile),
        out_type=[
            jax.ShapeDtypeStruct((n_rows,), jnp.float32),
            jax.ShapeDtypeStruct((n_rows,), jnp.float32),
        ],
        mesh=mesh,
        compiler_params=pltpu.CompilerParams(
            needs_layout_passes=False, use_tc_tiling_on_sc=True),
        scratch_types=[
            pltpu.VMEM((_D, _CW), jnp.float32),
            pltpu.VMEM((_D, _CW), jnp.float32),
            pltpu.VMEM((_D, _CW), jnp.float32),
            pltpu.VMEM((_D, _CW), jnp.float32),
            pltpu.VMEM((1, _CW), jnp.float32),
            pltpu.VMEM((1, _CW), jnp.float32),
            pltpu.VMEM((1, _CW), jnp.float32),
            pltpu.VMEM((1, _CW), jnp.float32),
            pltpu.VMEM((64,), jnp.float32),
            pltpu.VMEM((cols_per_tile,), jnp.float32),
            pltpu.VMEM((cols_per_tile,), jnp.float32),
            pltpu.SemaphoreType.DMA,
            pltpu.SemaphoreType.DMA,
        ],
    )(ut, it, ub_t, ib_t, wb)

    out = pl.kernel(
        functools.partial(_phase2, b_per_w),
        out_type=jax.ShapeDtypeStruct((batch,), jnp.float32),
        mesh=mesh,
        compiler_params=pltpu.CompilerParams(
            needs_layout_passes=False, use_tc_tiling_on_sc=False),
        scratch_types=[
            pltpu.VMEM((b_per_w,), jnp.int32),
            pltpu.VMEM((b_per_w,), jnp.int32),
            pltpu.VMEM((b_per_w,), jnp.float32),
            pltpu.VMEM((b_per_w,), jnp.float32),
            pltpu.VMEM((b_per_w,), jnp.float32),
            pltpu.VMEM((b_per_w,), jnp.float32),
            pltpu.VMEM((_LANES,), jnp.float32),
            pltpu.VMEM((b_per_w,), jnp.float32),
            pltpu.SemaphoreType.DMA,
        ],
    )(user_ids, item_ids, su, si, su2, si2, b16)
    return out.reshape(batch, 1)


# speedup vs baseline: 1.0744x; 1.0744x over previous
import functools

import jax
import jax.numpy as jnp
from jax import lax
from jax.experimental import pallas as pl
from jax.experimental.pallas import tpu as pltpu
from jax.experimental.pallas import tpu_sc as plsc

_D = 32
_LANES = 16
_BC = 49152
_T_SPLIT = 12
_SC_END = 983040
_CW = 768


def _phase1_tc(ut_ref, it_ref, ub_ref, ib_ref, wu_ref, wi_ref, su_ref, si_ref):
    su_ref[...] = jnp.sum(ut_ref[...] * wu_ref[...], axis=0) + ub_ref[0, :]
    si_ref[...] = jnp.sum(it_ref[...] * wi_ref[...], axis=0) + ib_ref[0, :]


def _phase1_sc(cols_per_tile,
               ut_hbm, it_hbm, ubt_hbm, ibt_hbm, wb_hbm,
               su_hbm, si_hbm,
               ubuf0, ubuf1, ibuf0, ibuf1, ubb0, ubb1, ibb0, ibb1,
               wb_v, ou_v, oi_v,
               sem0, sem1):
    wid = lax.axis_index("s") * 2 + lax.axis_index("c")
    base = _T_SPLIT * _BC + wid * cols_per_tile

    pltpu.sync_copy(wb_hbm, wb_v)
    wvecs = [wb_v[pl.ds(16 * k, 16)] for k in range(4)]

    def wsc(d):
        return wvecs[d // 16][d % 16]

    nch = cols_per_tile // _CW
    ubufs = [ubuf0, ubuf1]
    ibufs = [ibuf0, ibuf1]
    ubbs = [ubb0, ubb1]
    ibbs = [ibb0, ibb1]
    sems = [sem0, sem1]

    def start(c):
        c0 = base + c * _CW
        p = c % 2
        return [
            pltpu.async_copy(ut_hbm.at[:, pl.ds(c0, _CW)], ubufs[p], sems[p]),
            pltpu.async_copy(it_hbm.at[:, pl.ds(c0, _CW)], ibufs[p], sems[p]),
            pltpu.async_copy(ubt_hbm.at[:, pl.ds(c0, _CW)], ubbs[p], sems[p]),
            pltpu.async_copy(ibt_hbm.at[:, pl.ds(c0, _CW)], ibbs[p], sems[p]),
        ]

    pending = {0: start(0)}
    for c in range(nch):
        if c + 1 < nch:
            pending[c + 1] = start(c + 1)
        for cp in pending.pop(c):
            cp.wait()
        p = c % 2
        ub, ib, ubb, ibb = ubufs[p], ibufs[p], ubbs[p], ibbs[p]

        def grp(g, carry):
            o = g * _LANES
            pu = [ubb[0, pl.ds(o, _LANES)], None, None, None]
            pi = [ibb[0, pl.ds(o, _LANES)], None, None, None]
            for d in range(_D):
                k = d % 4
                t = ub[d, pl.ds(o, _LANES)] * wsc(d)
                pu[k] = t if pu[k] is None else pu[k] + t
            for d in range(_D):
                k = d % 4
                t = ib[d, pl.ds(o, _LANES)] * wsc(_D + d)
                pi[k] = t if pi[k] is None else pi[k] + t
            oo = c * _CW + o
            ou_v[pl.ds(oo, _LANES)] = (pu[0] + pu[1]) + (pu[2] + pu[3])
            oi_v[pl.ds(oo, _LANES)] = (pi[0] + pi[1]) + (pi[2] + pi[3])
            return carry

        lax.fori_loop(0, _CW // _LANES, grp, 0)

    pltpu.sync_copy(ou_v, su_hbm.at[pl.ds(base, cols_per_tile)])
    pltpu.sync_copy(oi_v, si_hbm.at[pl.ds(base, cols_per_tile)])


def _phase2(b_per_w,
            uid_hbm, iid_hbm, su_hbm, si_hbm, su2_hbm, si2_hbm, b16_hbm,
            out_hbm,
            uid_v, iid_v, sug_v, sig_v, sug2_v, sig2_v, b_v, out_v, sem):
    wid = lax.axis_index("s") * 2 + lax.axis_index("c")
    base = wid * b_per_w

    pltpu.sync_copy(uid_hbm.at[pl.ds(base, b_per_w)], uid_v)
    pltpu.sync_copy(iid_hbm.at[pl.ds(base, b_per_w)], iid_v)
    pltpu.sync_copy(b16_hbm, b_v)

    cps = [
        pltpu.async_copy(su_hbm.at[uid_v], sug_v, sem),
        pltpu.async_copy(si_hbm.at[iid_v], sig_v, sem),
        pltpu.async_copy(su2_hbm.at[uid_v], sug2_v, sem),
        pltpu.async_copy(si2_hbm.at[iid_v], sig2_v, sem),
    ]
    for cp in cps:
        cp.wait()

    b_vec = b_v[...]
    lo = _T_SPLIT * _BC

    def blk(i, carry):
        rbase = i * _LANES
        sl = pl.ds(rbase, _LANES)
        uid = uid_v[sl]
        iid = iid_v[sl]
        u_tc = jnp.logical_or(uid < lo, uid >= _SC_END)
        i_tc = jnp.logical_or(iid < lo, iid >= _SC_END)
        su = jnp.where(u_tc, sug_v[sl], sug2_v[sl])
        si = jnp.where(i_tc, sig_v[sl], sig2_v[sl])
        out_v[sl] = jnp.clip(su + si + b_vec, 0.5, 5.0)
        return carry

    lax.fori_loop(0, b_per_w // _LANES, blk, 0)

    pltpu.sync_copy(out_v, out_hbm.at[pl.ds(base, b_per_w)])


def kernel(user_ids, item_ids, user_emb, item_emb, user_bias_tab, item_bias_tab, W, b):
    batch = user_ids.shape[0]
    n_workers = 32
    b_per_w = batch // n_workers
    n_rows = user_emb.shape[0]

    ut = user_emb.T
    it = item_emb.T
    ub_t = user_bias_tab.T
    ib_t = item_bias_tab.T
    wu = W[0, :_D].reshape(_D, 1)
    wi = W[0, _D:].reshape(_D, 1)
    wb = jnp.zeros((64,), jnp.float32).at[:2 * _D].set(W.reshape(-1))
    b16 = jnp.full((_LANES,), b[0], jnp.float32)

    nb = (n_rows + _BC - 1) // _BC
    tc_grid = _T_SPLIT + 1

    def colmap(j):
        return (0, jnp.where(j < _T_SPLIT, j, nb - 1))

    def outmap(j):
        return (jnp.where(j < _T_SPLIT, j, nb - 1),)

    su, si = pl.pallas_call(
        _phase1_tc,
        grid=(tc_grid,),
        in_specs=[
            pl.BlockSpec((_D, _BC), colmap),
            pl.BlockSpec((_D, _BC), colmap),
            pl.BlockSpec((1, _BC), colmap),
            pl.BlockSpec((1, _BC), colmap),
            pl.BlockSpec((_D, 1), lambda j: (0, 0)),
            pl.BlockSpec((_D, 1), lambda j: (0, 0)),
        ],
        out_specs=[
            pl.BlockSpec((_BC,), outmap),
            pl.BlockSpec((_BC,), outmap),
        ],
        out_shape=[
            jax.ShapeDtypeStruct((n_rows,), jnp.float32),
            jax.ShapeDtypeStruct((n_rows,), jnp.float32),
        ],
    )(ut, it, ub_t, ib_t, wu, wi)

    cols_per_tile = (_SC_END - _T_SPLIT * _BC) // n_workers
    mesh = plsc.VectorSubcoreMesh(core_axis_name="c", subcore_axis_name="s")
    su2, si2 = pl.kernel(
        functools.partial(_phase1_sc, cols_per_tile),
        out_type=[
            jax.ShapeDtypeStruct((n_rows,), jnp.float32),
            jax.ShapeDtypeStruct((n_rows,), jnp.float32),
        ],
        mesh=mesh,
        compiler_params=pltpu.CompilerParams(
            needs_layout_passes=False, use_tc_tiling_on_sc=True),
        scratch_types=[
            pltpu.VMEM((_D, _CW), jnp.float32),
            pltpu.VMEM((_D, _CW), jnp.float32),
            pltpu.VMEM((_D, _CW), jnp.float32),
            pltpu.VMEM((_D, _CW), jnp.float32),
            pltpu.VMEM((1, _CW), jnp.float32),
            pltpu.VMEM((1, _CW), jnp.float32),
            pltpu.VMEM((1, _CW), jnp.float32),
            pltpu.VMEM((1, _CW), jnp.float32),
            pltpu.VMEM((64,), jnp.float32),
            pltpu.VMEM((cols_per_tile,), jnp.float32),
            pltpu.VMEM((cols_per_tile,), jnp.float32),
            pltpu.SemaphoreType.DMA,
            pltpu.SemaphoreType.DMA,
        ],
    )(ut, it, ub_t, ib_t, wb)

    out = pl.kernel(
        functools.partial(_phase2, b_per_w),
        out_type=jax.ShapeDtypeStruct((batch,), jnp.float32),
        mesh=mesh,
        compiler_params=pltpu.CompilerParams(
            needs_layout_passes=False, use_tc_tiling_on_sc=False),
        scratch_types=[
            pltpu.VMEM((b_per_w,), jnp.int32),
            pltpu.VMEM((b_per_w,), jnp.int32),
            pltpu.VMEM((b_per_w,), jnp.float32),
            pltpu.VMEM((b_per_w,), jnp.float32),
            pltpu.VMEM((b_per_w,), jnp.float32),
            pltpu.VMEM((b_per_w,), jnp.float32),
            pltpu.VMEM((_LANES,), jnp.float32),
            pltpu.VMEM((b_per_w,), jnp.float32),
            pltpu.SemaphoreType.DMA,
        ],
    )(user_ids, item_ids, su, si, su2, si2, b16)
    return out.reshape(batch, 1)


# speedup vs baseline: 1.0820x; 1.0071x over previous
import functools

import jax
import jax.numpy as jnp
from jax import lax
from jax.experimental import pallas as pl
from jax.experimental.pallas import tpu as pltpu
from jax.experimental.pallas import tpu_sc as plsc

_D = 32
_LANES = 16
_BC = 49152
_T_SPLIT = 14
_SC_END = 983040
_CW = 768


def _phase1_tc(ut_ref, it_ref, ub_ref, ib_ref, wu_ref, wi_ref, su_ref, si_ref):
    su_ref[...] = jnp.sum(ut_ref[...] * wu_ref[...], axis=0) + ub_ref[0, :]
    si_ref[...] = jnp.sum(it_ref[...] * wi_ref[...], axis=0) + ib_ref[0, :]


def _phase1_sc(cols_per_tile,
               ut_hbm, it_hbm, ubt_hbm, ibt_hbm, wb_hbm,
               su_hbm, si_hbm,
               ubuf0, ubuf1, ibuf0, ibuf1, ubb0, ubb1, ibb0, ibb1,
               wb_v, ou_v, oi_v,
               sem0, sem1):
    wid = lax.axis_index("s") * 2 + lax.axis_index("c")
    base = _T_SPLIT * _BC + wid * cols_per_tile

    pltpu.sync_copy(wb_hbm, wb_v)
    wvecs = [wb_v[pl.ds(16 * k, 16)] for k in range(4)]

    def wsc(d):
        return wvecs[d // 16][d % 16]

    nch = cols_per_tile // _CW
    ubufs = [ubuf0, ubuf1]
    ibufs = [ibuf0, ibuf1]
    ubbs = [ubb0, ubb1]
    ibbs = [ibb0, ibb1]
    sems = [sem0, sem1]

    def start(c):
        c0 = base + c * _CW
        p = c % 2
        return [
            pltpu.async_copy(ut_hbm.at[:, pl.ds(c0, _CW)], ubufs[p], sems[p]),
            pltpu.async_copy(it_hbm.at[:, pl.ds(c0, _CW)], ibufs[p], sems[p]),
            pltpu.async_copy(ubt_hbm.at[:, pl.ds(c0, _CW)], ubbs[p], sems[p]),
            pltpu.async_copy(ibt_hbm.at[:, pl.ds(c0, _CW)], ibbs[p], sems[p]),
        ]

    pending = {0: start(0)}
    for c in range(nch):
        if c + 1 < nch:
            pending[c + 1] = start(c + 1)
        for cp in pending.pop(c):
            cp.wait()
        p = c % 2
        ub, ib, ubb, ibb = ubufs[p], ibufs[p], ubbs[p], ibbs[p]

        def grp(g, carry):
            o = g * _LANES
            pu = [ubb[0, pl.ds(o, _LANES)], None, None, None]
            pi = [ibb[0, pl.ds(o, _LANES)], None, None, None]
            for d in range(_D):
                k = d % 4
                t = ub[d, pl.ds(o, _LANES)] * wsc(d)
                pu[k] = t if pu[k] is None else pu[k] + t
            for d in range(_D):
                k = d % 4
                t = ib[d, pl.ds(o, _LANES)] * wsc(_D + d)
                pi[k] = t if pi[k] is None else pi[k] + t
            oo = c * _CW + o
            ou_v[pl.ds(oo, _LANES)] = (pu[0] + pu[1]) + (pu[2] + pu[3])
            oi_v[pl.ds(oo, _LANES)] = (pi[0] + pi[1]) + (pi[2] + pi[3])
            return carry

        lax.fori_loop(0, _CW // _LANES, grp, 0)

    pltpu.sync_copy(ou_v, su_hbm.at[pl.ds(base, cols_per_tile)])
    pltpu.sync_copy(oi_v, si_hbm.at[pl.ds(base, cols_per_tile)])


def _phase2(b_per_w,
            uid_hbm, iid_hbm, su_hbm, si_hbm, su2_hbm, si2_hbm, b16_hbm,
            out_hbm,
            uid_v, iid_v, sug_v, sig_v, sug2_v, sig2_v, b_v, out_v, sem):
    wid = lax.axis_index("s") * 2 + lax.axis_index("c")
    base = wid * b_per_w

    pltpu.sync_copy(uid_hbm.at[pl.ds(base, b_per_w)], uid_v)
    pltpu.sync_copy(iid_hbm.at[pl.ds(base, b_per_w)], iid_v)
    pltpu.sync_copy(b16_hbm, b_v)

    cps = [
        pltpu.async_copy(su_hbm.at[uid_v], sug_v, sem),
        pltpu.async_copy(si_hbm.at[iid_v], sig_v, sem),
        pltpu.async_copy(su2_hbm.at[uid_v], sug2_v, sem),
        pltpu.async_copy(si2_hbm.at[iid_v], sig2_v, sem),
    ]
    for cp in cps:
        cp.wait()

    b_vec = b_v[...]
    lo = _T_SPLIT * _BC

    def blk(i, carry):
        rbase = i * _LANES
        sl = pl.ds(rbase, _LANES)
        uid = uid_v[sl]
        iid = iid_v[sl]
        u_tc = jnp.logical_or(uid < lo, uid >= _SC_END)
        i_tc = jnp.logical_or(iid < lo, iid >= _SC_END)
        su = jnp.where(u_tc, sug_v[sl], sug2_v[sl])
        si = jnp.where(i_tc, sig_v[sl], sig2_v[sl])
        out_v[sl] = jnp.clip(su + si + b_vec, 0.5, 5.0)
        return carry

    lax.fori_loop(0, b_per_w // _LANES, blk, 0)

    pltpu.sync_copy(out_v, out_hbm.at[pl.ds(base, b_per_w)])


def kernel(user_ids, item_ids, user_emb, item_emb, user_bias_tab, item_bias_tab, W, b):
    batch = user_ids.shape[0]
    n_workers = 32
    b_per_w = batch // n_workers
    n_rows = user_emb.shape[0]

    ut = user_emb.T
    it = item_emb.T
    ub_t = user_bias_tab.T
    ib_t = item_bias_tab.T
    wu = W[0, :_D].reshape(_D, 1)
    wi = W[0, _D:].reshape(_D, 1)
    wb = jnp.zeros((64,), jnp.float32).at[:2 * _D].set(W.reshape(-1))
    b16 = jnp.full((_LANES,), b[0], jnp.float32)

    nb = (n_rows + _BC - 1) // _BC
    tc_grid = _T_SPLIT + 1

    def colmap(j):
        return (0, jnp.where(j < _T_SPLIT, j, nb - 1))

    def outmap(j):
        return (jnp.where(j < _T_SPLIT, j, nb - 1),)

    su, si = pl.pallas_call(
        _phase1_tc,
        grid=(tc_grid,),
        in_specs=[
            pl.BlockSpec((_D, _BC), colmap),
            pl.BlockSpec((_D, _BC), colmap),
            pl.BlockSpec((1, _BC), colmap),
            pl.BlockSpec((1, _BC), colmap),
            pl.BlockSpec((_D, 1), lambda j: (0, 0)),
            pl.BlockSpec((_D, 1), lambda j: (0, 0)),
        ],
        out_specs=[
            pl.BlockSpec((_BC,), outmap),
            pl.BlockSpec((_BC,), outmap),
        ],
        out_shape=[
            jax.ShapeDtypeStruct((n_rows,), jnp.float32),
            jax.ShapeDtypeStruct((n_rows,), jnp.float32),
        ],
    )(ut, it, ub_t, ib_t, wu, wi)

    cols_per_tile = (_SC_END - _T_SPLIT * _BC) // n_workers
    mesh = plsc.VectorSubcoreMesh(core_axis_name="c", subcore_axis_name="s")
    su2, si2 = pl.kernel(
        functools.partial(_phase1_sc, cols_per_tile),
        out_type=[
            jax.ShapeDtypeStruct((n_rows,), jnp.float32),
            jax.ShapeDtypeStruct((n_rows,), jnp.float32),
        ],
        mesh=mesh,
        compiler_params=pltpu.CompilerParams(
            needs_layout_passes=False, use_tc_tiling_on_sc=True),
        scratch_types=[
            pltpu.VMEM((_D, _CW), jnp.float32),
            pltpu.VMEM((_D, _CW), jnp.float32),
            pltpu.VMEM((_D, _CW), jnp.float32),
            pltpu.VMEM((_D, _CW), jnp.float32),
            pltpu.VMEM((1, _CW), jnp.float32),
            pltpu.VMEM((1, _CW), jnp.float32),
            pltpu.VMEM((1, _CW), jnp.float32),
            pltpu.VMEM((1, _CW), jnp.float32),
            pltpu.VMEM((64,), jnp.float32),
            pltpu.VMEM((cols_per_tile,), jnp.float32),
            pltpu.VMEM((cols_per_tile,), jnp.float32),
            pltpu.SemaphoreType.DMA,
            pltpu.SemaphoreType.DMA,
        ],
    )(ut, it, ub_t, ib_t, wb)

    out = pl.kernel(
        functools.partial(_phase2, b_per_w),
        out_type=jax.ShapeDtypeStruct((batch,), jnp.float32),
        mesh=mesh,
        compiler_params=pltpu.CompilerParams(
            needs_layout_passes=False, use_tc_tiling_on_sc=False),
        scratch_types=[
            pltpu.VMEM((b_per_w,), jnp.int32),
            pltpu.VMEM((b_per_w,), jnp.int32),
            pltpu.VMEM((b_per_w,), jnp.float32),
            pltpu.VMEM((b_per_w,), jnp.float32),
            pltpu.VMEM((b_per_w,), jnp.float32),
            pltpu.VMEM((b_per_w,), jnp.float32),
            pltpu.VMEM((_LANES,), jnp.float32),
            pltpu.VMEM((b_per_w,), jnp.float32),
            pltpu.SemaphoreType.DMA,
        ],
    )(user_ids, item_ids, su, si, su2, si2, b16)
    return out.reshape(batch, 1)


# speedup vs baseline: 1.1038x; 1.0201x over previous
import functools

import jax
import jax.numpy as jnp
from jax import lax
from jax.experimental import pallas as pl
from jax.experimental.pallas import tpu as pltpu
from jax.experimental.pallas import tpu_sc as plsc

_D = 32
_LANES = 16
_BC = 57344


def _phase1(ut_ref, it_ref, ub_ref, ib_ref, wu_ref, wi_ref, su_ref, si_ref):
    su_ref[...] = jnp.sum(ut_ref[...] * wu_ref[...], axis=0) + ub_ref[0, :]
    si_ref[...] = jnp.sum(it_ref[...] * wi_ref[...], axis=0) + ib_ref[0, :]


def _phase2(b_per_w,
            uid_hbm, iid_hbm, su_hbm, si_hbm, b16_hbm,
            out_hbm,
            uid_v, iid_v, sug_v, sig_v, b_v, out_v, sem):
    wid = lax.axis_index("s") * 2 + lax.axis_index("c")
    base = wid * b_per_w

    pltpu.sync_copy(uid_hbm.at[pl.ds(base, b_per_w)], uid_v)
    pltpu.sync_copy(iid_hbm.at[pl.ds(base, b_per_w)], iid_v)
    pltpu.sync_copy(b16_hbm, b_v)

    cp_u = pltpu.async_copy(su_hbm.at[uid_v], sug_v, sem)
    cp_i = pltpu.async_copy(si_hbm.at[iid_v], sig_v, sem)
    cp_u.wait()
    cp_i.wait()

    b_vec = b_v[...]

    def blk(i, carry):
        rbase = i * _LANES
        acc = sug_v[pl.ds(rbase, _LANES)] + sig_v[pl.ds(rbase, _LANES)]
        out_v[pl.ds(rbase, _LANES)] = jnp.clip(acc + b_vec, 0.5, 5.0)
        return carry

    lax.fori_loop(0, b_per_w // _LANES, blk, 0)

    pltpu.sync_copy(out_v, out_hbm.at[pl.ds(base, b_per_w)])


def kernel(user_ids, item_ids, user_emb, item_emb, user_bias_tab, item_bias_tab, W, b):
    batch = user_ids.shape[0]
    n_workers = 32
    b_per_w = batch // n_workers
    n_rows = user_emb.shape[0]

    ut = user_emb.T
    it = item_emb.T
    ub_t = user_bias_tab.T
    ib_t = item_bias_tab.T
    wu = W[0, :_D].reshape(_D, 1)
    wi = W[0, _D:].reshape(_D, 1)
    b16 = jnp.full((_LANES,), b[0], jnp.float32)

    nb = (n_rows + _BC - 1) // _BC
    su, si = pl.pallas_call(
        _phase1,
        grid=(nb,),
        in_specs=[
            pl.BlockSpec((_D, _BC), lambda i: (0, i)),
            pl.BlockSpec((_D, _BC), lambda i: (0, i)),
            pl.BlockSpec((1, _BC), lambda i: (0, i)),
            pl.BlockSpec((1, _BC), lambda i: (0, i)),
            pl.BlockSpec((_D, 1), lambda i: (0, 0)),
            pl.BlockSpec((_D, 1), lambda i: (0, 0)),
        ],
        out_specs=[
            pl.BlockSpec((_BC,), lambda i: (i,)),
            pl.BlockSpec((_BC,), lambda i: (i,)),
        ],
        out_shape=[
            jax.ShapeDtypeStruct((n_rows,), jnp.float32),
            jax.ShapeDtypeStruct((n_rows,), jnp.float32),
        ],
    )(ut, it, ub_t, ib_t, wu, wi)

    mesh = plsc.VectorSubcoreMesh(core_axis_name="c", subcore_axis_name="s")
    out = pl.kernel(
        functools.partial(_phase2, b_per_w),
        out_type=jax.ShapeDtypeStruct((batch,), jnp.float32),
        mesh=mesh,
        compiler_params=pltpu.CompilerParams(
            needs_layout_passes=False, use_tc_tiling_on_sc=False),
        scratch_types=[
            pltpu.VMEM((b_per_w,), jnp.int32),
            pltpu.VMEM((b_per_w,), jnp.int32),
            pltpu.VMEM((b_per_w,), jnp.float32),
            pltpu.VMEM((b_per_w,), jnp.float32),
            pltpu.VMEM((_LANES,), jnp.float32),
            pltpu.VMEM((b_per_w,), jnp.float32),
            pltpu.SemaphoreType.DMA,
        ],
    )(user_ids, item_ids, su, si, b16)
    return out.reshape(batch, 1)


# speedup vs baseline: 1.1524x; 1.0440x over previous
import functools

import jax
import jax.numpy as jnp
from jax import lax
from jax.experimental import pallas as pl
from jax.experimental.pallas import tpu as pltpu
from jax.experimental.pallas import tpu_sc as plsc

_D = 32
_LANES = 16
_BC = 49152


def _phase1(ut_ref, it_ref, ub_ref, ib_ref, wu_ref, wi_ref, su_ref, si_ref):
    su_ref[...] = (jnp.dot(wu_ref[...], ut_ref[...],
                           preferred_element_type=jnp.float32)
                   + ub_ref[...])[0, :]
    si_ref[...] = (jnp.dot(wi_ref[...], it_ref[...],
                           preferred_element_type=jnp.float32)
                   + ib_ref[...])[0, :]


def _phase2(b_per_w,
            uid_hbm, iid_hbm, su_hbm, si_hbm, b16_hbm,
            out_hbm,
            uid_v, iid_v, sug_v, sig_v, b_v, out_v, sem):
    wid = lax.axis_index("s") * 2 + lax.axis_index("c")
    base = wid * b_per_w

    pltpu.sync_copy(uid_hbm.at[pl.ds(base, b_per_w)], uid_v)
    pltpu.sync_copy(iid_hbm.at[pl.ds(base, b_per_w)], iid_v)
    pltpu.sync_copy(b16_hbm, b_v)

    cp_u = pltpu.async_copy(su_hbm.at[uid_v], sug_v, sem)
    cp_i = pltpu.async_copy(si_hbm.at[iid_v], sig_v, sem)
    cp_u.wait()
    cp_i.wait()

    b_vec = b_v[...]

    def blk(i, carry):
        rbase = i * _LANES
        acc = sug_v[pl.ds(rbase, _LANES)] + sig_v[pl.ds(rbase, _LANES)]
        out_v[pl.ds(rbase, _LANES)] = jnp.clip(acc + b_vec, 0.5, 5.0)
        return carry

    lax.fori_loop(0, b_per_w // _LANES, blk, 0)

    pltpu.sync_copy(out_v, out_hbm.at[pl.ds(base, b_per_w)])


def kernel(user_ids, item_ids, user_emb, item_emb, user_bias_tab, item_bias_tab, W, b):
    batch = user_ids.shape[0]
    n_workers = 32
    b_per_w = batch // n_workers
    n_rows = user_emb.shape[0]

    ut = user_emb.T
    it = item_emb.T
    ub_t = user_bias_tab.T
    ib_t = item_bias_tab.T
    wu = W[0, :_D].reshape(1, _D)
    wi = W[0, _D:].reshape(1, _D)
    b16 = jnp.full((_LANES,), b[0], jnp.float32)

    nb = (n_rows + _BC - 1) // _BC
    su, si = pl.pallas_call(
        _phase1,
        grid=(nb,),
        in_specs=[
            pl.BlockSpec((_D, _BC), lambda i: (0, i)),
            pl.BlockSpec((_D, _BC), lambda i: (0, i)),
            pl.BlockSpec((1, _BC), lambda i: (0, i)),
            pl.BlockSpec((1, _BC), lambda i: (0, i)),
            pl.BlockSpec((1, _D), lambda i: (0, 0)),
            pl.BlockSpec((1, _D), lambda i: (0, 0)),
        ],
        out_specs=[
            pl.BlockSpec((_BC,), lambda i: (i,)),
            pl.BlockSpec((_BC,), lambda i: (i,)),
        ],
        out_shape=[
            jax.ShapeDtypeStruct((n_rows,), jnp.float32),
            jax.ShapeDtypeStruct((n_rows,), jnp.float32),
        ],
    )(ut, it, ub_t, ib_t, wu, wi)

    mesh = plsc.VectorSubcoreMesh(core_axis_name="c", subcore_axis_name="s")
    out = pl.kernel(
        functools.partial(_phase2, b_per_w),
        out_type=jax.ShapeDtypeStruct((batch,), jnp.float32),
        mesh=mesh,
        compiler_params=pltpu.CompilerParams(
            needs_layout_passes=False, use_tc_tiling_on_sc=False),
        scratch_types=[
            pltpu.VMEM((b_per_w,), jnp.int32),
            pltpu.VMEM((b_per_w,), jnp.int32),
            pltpu.VMEM((b_per_w,), jnp.float32),
            pltpu.VMEM((b_per_w,), jnp.float32),
            pltpu.VMEM((_LANES,), jnp.float32),
            pltpu.VMEM((b_per_w,), jnp.float32),
            pltpu.SemaphoreType.DMA,
        ],
    )(user_ids, item_ids, su, si, b16)
    return out.reshape(batch, 1)
